# Initial kernel scaffold; baseline (speedup 1.0000x reference)
import jax
import jax.numpy as jnp
from jax.experimental import pallas as pl


def kernel(aff_idx, mat_idx, aff_table, mat_table, W, b):
    raise NotImplementedError("write your pallas kernel here")


# speedup vs baseline: 1.1911x; 1.1911x over previous
import functools

import jax
import jax.numpy as jnp
from jax import lax
from jax.experimental import pallas as pl
from jax.experimental.pallas import tpu as pltpu
from jax.experimental.pallas import tpu_sc as plsc

B = 16384
DA = 64
DM = 64
DOUT = 128
NC = 2
NS = 16
NW = NC * NS
B_PER_W = B // NW
CHUNK = 128
NCHUNK = B_PER_W // CHUNK


def _sc_gather(aff_idx_r, mat_idx_r, aff_table, mat_table):
    mesh = plsc.VectorSubcoreMesh(core_axis_name="c", subcore_axis_name="s")

    @functools.partial(
        pl.kernel,
        mesh=mesh,
        compiler_params=pltpu.CompilerParams(use_tc_tiling_on_sc=False),
        out_type=[
            jax.ShapeDtypeStruct((B, DA), jnp.float32),
            jax.ShapeDtypeStruct((B, DM), jnp.float32),
        ],
        scratch_types=[
            pltpu.VMEM((NCHUNK, CHUNK), jnp.int32),
            pltpu.VMEM((NCHUNK, CHUNK), jnp.int32),
            pltpu.VMEM((B_PER_W, DA), jnp.float32),
            pltpu.VMEM((B_PER_W, DM), jnp.float32),
            pltpu.SemaphoreType.DMA,
        ],
    )
    def k(aff_idx_hbm, mat_idx_hbm, aff_t_hbm, mat_t_hbm,
          out_a_hbm, out_m_hbm, aidx_v, midx_v, arows_v, mrows_v, sem):
        wid = lax.axis_index("s") * NC + lax.axis_index("c")
        base = wid * B_PER_W
        pltpu.sync_copy(aff_idx_hbm.at[wid], aidx_v)
        pltpu.sync_copy(mat_idx_hbm.at[wid], midx_v)
        copies = []
        for j in range(NCHUNK):
            copies.append(pltpu.async_copy(
                aff_t_hbm.at[aidx_v.at[j]],
                arows_v.at[pl.ds(j * CHUNK, CHUNK)], sem))
            copies.append(pltpu.async_copy(
                mat_t_hbm.at[midx_v.at[j]],
                mrows_v.at[pl.ds(j * CHUNK, CHUNK)], sem))
        for c in copies:
            c.wait()
        pltpu.sync_copy(arows_v, out_a_hbm.at[pl.ds(base, B_PER_W)])
        pltpu.sync_copy(mrows_v, out_m_hbm.at[pl.ds(base, B_PER_W)])

    return k(aff_idx_r, mat_idx_r, aff_table, mat_table)


_BT = 2048


def _tc_body(a_ref, m_ref, w1_ref, w2_ref, b_ref, out_ref):
    acc = jnp.dot(a_ref[...], w1_ref[...], preferred_element_type=jnp.float32)
    acc += jnp.dot(m_ref[...], w2_ref[...], preferred_element_type=jnp.float32)
    out_ref[...] = jnp.maximum(acc + b_ref[...], 0.0)


def _tc_linear(aff_e, mat_e, w1t, w2t, b2d):
    grid = (B // _BT,)
    return pl.pallas_call(
        _tc_body,
        grid=grid,
        in_specs=[
            pl.BlockSpec((_BT, DA), lambda i: (i, 0)),
            pl.BlockSpec((_BT, DM), lambda i: (i, 0)),
            pl.BlockSpec((DA, DOUT), lambda i: (0, 0)),
            pl.BlockSpec((DM, DOUT), lambda i: (0, 0)),
            pl.BlockSpec((1, DOUT), lambda i: (0, 0)),
        ],
        out_specs=pl.BlockSpec((_BT, DOUT), lambda i: (i, 0)),
        out_shape=jax.ShapeDtypeStruct((B, DOUT), jnp.float32),
    )(aff_e, mat_e, w1t, w2t, b2d)


def kernel(aff_idx, mat_idx, aff_table, mat_table, W, b):
    aff_idx_r = aff_idx.astype(jnp.int32).reshape(NW, NCHUNK, CHUNK)
    mat_idx_r = mat_idx.astype(jnp.int32).reshape(NW, NCHUNK, CHUNK)
    aff_e, mat_e = _sc_gather(aff_idx_r, mat_idx_r, aff_table, mat_table)
    w1t = W[:, :DA].T
    w2t = W[:, DA:].T
    return _tc_linear(aff_e, mat_e, w1t, w2t, b.reshape(1, DOUT))
